# Initial kernel scaffold; baseline (speedup 1.0000x reference)
import functools
import math

import jax
import jax.numpy as jnp
from jax import lax
from jax.experimental import pallas as pl
from jax.experimental.pallas import tpu as pltpu
from jax.experimental.pallas import tpu_sc as plsc

_NC = 2
_NS = 16
_NW = _NC * _NS
_KROWS = 16
_SC_EDGES = _KROWS * 128


def _stripe(n):
    r0 = (n // _NS // 8) * 8
    return r0, n - (_NS - 1) * r0


@functools.lru_cache(maxsize=None)
def _make_edge_pass(n, e_pad, mode):
    nsup = e_pad // (_NW * _SC_EDGES)
    t_edges = nsup * _SC_EDGES
    r0, r_last = _stripe(n)
    acc_shape = (n, 16) if mode == "vec16" else (n,)
    out_shape = (_NC * n, 16) if mode == "vec16" else (_NC * n,)

    mesh = plsc.VectorSubcoreMesh(
        core_axis_name="c", subcore_axis_name="s",
        num_cores=_NC, num_subcores=_NS)

    scratch = []
    if mode == "flat1":
        scratch.append(pltpu.VMEM((n,), jnp.float32))
        scratch.append(pltpu.VMEM((_SC_EDGES,), jnp.int32))
    if mode == "vec16":
        scratch.append(pltpu.VMEM((_KROWS, 128), jnp.int32))
    scratch.append(pltpu.VMEM((_KROWS, 128), jnp.int32))
    scratch.append(pltpu.VMEM((_SC_EDGES,), jnp.float32))
    if mode != "deg":
        msg_shape = (_SC_EDGES, 16) if mode == "vec16" else (_SC_EDGES,)
        scratch.append(pltpu.VMEM(msg_shape, jnp.float32))
    scratch.append(pltpu.VMEM_SHARED(acc_shape, jnp.float32))
    scratch.append(pltpu.SemaphoreType.DMA)
    scratch.append(pltpu.SemaphoreType.DMA)

    def body(*refs):
        it = iter(refs)
        u_hbm = row_hbm = None
        if mode in ("vec16", "flat1"):
            u_hbm = next(it)
            row_hbm = next(it)
        col_hbm = next(it)
        w_hbm = next(it)
        z_hbm = next(it)
        out_hbm = next(it)
        u_v = idx_row = None
        if mode == "flat1":
            u_v = next(it)
            idx_row = next(it)
        elif mode == "vec16":
            idx_row = next(it)
        idx_col = next(it)
        w_v = next(it)
        msg_v = next(it) if mode != "deg" else None
        acc = next(it)
        gsem = next(it)
        ssem = next(it)

        c = lax.axis_index("c")
        s = lax.axis_index("s")
        g = c * _NS + s

        @pl.when(s < _NS - 1)
        def _():
            pltpu.sync_copy(z_hbm.at[pl.ds(s * r0, r0)],
                            acc.at[pl.ds(s * r0, r0)])

        @pl.when(s == _NS - 1)
        def _():
            pltpu.sync_copy(z_hbm.at[pl.ds((_NS - 1) * r0, r_last)],
                            acc.at[pl.ds((_NS - 1) * r0, r_last)])

        if mode == "flat1":
            pltpu.sync_copy(u_hbm, u_v)
        plsc.subcore_barrier()

        @pl.loop(0, nsup)
        def _sup(i):
            e_base = g * t_edges + i * _SC_EDGES
            r_base = (g * t_edges) // 128 + i * _KROWS
            pltpu.sync_copy(col_hbm.at[pl.ds(r_base, _KROWS)], idx_col)
            pltpu.sync_copy(w_hbm.at[pl.ds(e_base, _SC_EDGES)], w_v)
            if mode == "vec16":
                pltpu.sync_copy(row_hbm.at[pl.ds(r_base, _KROWS)], idx_row)
                descs = [
                    pltpu.async_copy(u_hbm.at[idx_row.at[j]],
                                     msg_v.at[pl.ds(j * 128, 128)], gsem)
                    for j in range(_KROWS)]
                for d in descs:
                    d.wait()

                @pl.loop(0, _SC_EDGES, unroll=8)
                def _e(e):
                    msg_v[e, :] = msg_v[e, :] * w_v[e]

            elif mode == "flat1":
                pltpu.sync_copy(row_hbm.at[pl.ds(e_base, _SC_EDGES)], idx_row)

                @pl.loop(0, _SC_EDGES // 16, unroll=8)
                def _grp(q):
                    idx16 = idx_row[pl.ds(q * 16, 16)]
                    rows16 = plsc.load_gather(u_v, [idx16])
                    msg_v[pl.ds(q * 16, 16)] = (
                        rows16 * w_v[pl.ds(q * 16, 16)])

            src = w_v if mode == "deg" else msg_v
            sdescs = [
                pltpu.async_copy(src.at[pl.ds(j * 128, 128)],
                                 acc.at[idx_col.at[j]], ssem, add=True)
                for j in range(_KROWS)]
            for d in sdescs:
                d.wait()

        plsc.subcore_barrier()

        @pl.when(s < _NS - 1)
        def _():
            pltpu.sync_copy(acc.at[pl.ds(s * r0, r0)],
                            out_hbm.at[pl.ds(c * n + s * r0, r0)])

        @pl.when(s == _NS - 1)
        def _():
            pltpu.sync_copy(
                acc.at[pl.ds((_NS - 1) * r0, r_last)],
                out_hbm.at[pl.ds(c * n + (_NS - 1) * r0, r_last)])

    return pl.kernel(
        body,
        out_type=jax.ShapeDtypeStruct(out_shape, jnp.float32),
        mesh=mesh,
        scratch_types=scratch)


_BN = 1000
_CBN = 1.0 / math.sqrt(1.0 + 1e-5)


def _tc_first(x, w1, p0, p1):
    n, f = x.shape

    def body(x_ref, w_ref, p0_ref, p1_ref, t_ref, u_ref, dinv_ref):
        deg = p0_ref[...] + p1_ref[...] + 1.0
        dinv = lax.rsqrt(deg)
        t = jnp.dot(x_ref[...], w_ref[...],
                    preferred_element_type=jnp.float32)
        t_ref[...] = t
        u_ref[...] = t * dinv
        dinv_ref[...] = dinv

    return pl.pallas_call(
        body,
        grid=(n // _BN,),
        in_specs=[
            pl.BlockSpec((_BN, f), lambda i: (i, 0)),
            pl.BlockSpec((f, 16), lambda i: (0, 0)),
            pl.BlockSpec((_BN, 1), lambda i: (i, 0)),
            pl.BlockSpec((_BN, 1), lambda i: (i, 0)),
        ],
        out_specs=[
            pl.BlockSpec((_BN, 16), lambda i: (i, 0)),
            pl.BlockSpec((_BN, 16), lambda i: (i, 0)),
            pl.BlockSpec((_BN, 1), lambda i: (i, 0)),
        ],
        out_shape=[
            jax.ShapeDtypeStruct((n, 16), jnp.float32),
            jax.ShapeDtypeStruct((n, 16), jnp.float32),
            jax.ShapeDtypeStruct((n, 1), jnp.float32),
        ],
    )(x, w1, p0, p1)


def _tc_mid(sp0, sp1, t, dinv, b, gam, bt, wnext):
    n = t.shape[0]

    def body(p0_ref, p1_ref, t_ref, dinv_ref, b_ref, g_ref, bt_ref, w_ref,
             tn_ref, un_ref):
        dinv = dinv_ref[...]
        pre = (dinv * (p0_ref[...] + p1_ref[...])
               + (dinv * dinv) * t_ref[...] + b_ref[...])
        h = jnp.maximum(g_ref[...] * (pre * _CBN) + bt_ref[...], 0.0)
        tn = jnp.dot(h, w_ref[...], preferred_element_type=jnp.float32)
        tn_ref[...] = tn
        un_ref[...] = tn * dinv

    return pl.pallas_call(
        body,
        grid=(n // _BN,),
        in_specs=[
            pl.BlockSpec((_BN, 16), lambda i: (i, 0)),
            pl.BlockSpec((_BN, 16), lambda i: (i, 0)),
            pl.BlockSpec((_BN, 16), lambda i: (i, 0)),
            pl.BlockSpec((_BN, 1), lambda i: (i, 0)),
            pl.BlockSpec((1, 16), lambda i: (0, 0)),
            pl.BlockSpec((1, 16), lambda i: (0, 0)),
            pl.BlockSpec((1, 16), lambda i: (0, 0)),
            pl.BlockSpec((16, 16), lambda i: (0, 0)),
        ],
        out_specs=[
            pl.BlockSpec((_BN, 16), lambda i: (i, 0)),
            pl.BlockSpec((_BN, 16), lambda i: (i, 0)),
        ],
        out_shape=[
            jax.ShapeDtypeStruct((n, 16), jnp.float32),
            jax.ShapeDtypeStruct((n, 16), jnp.float32),
        ],
    )(sp0, sp1, t, dinv, b, gam, bt, wnext)


def _tc_final(sp0, sp1, t3, dinv, b3, g3, bt3, wl, bl):
    n = t3.shape[0]

    def body(p0_ref, p1_ref, t_ref, dinv_ref, b_ref, g_ref, bt_ref,
             wl_ref, bl_ref, o_ref):
        dinv = dinv_ref[...]
        pre = (dinv * (p0_ref[...] + p1_ref[...])
               + (dinv * dinv) * t_ref[...] + b_ref[...])
        h3 = g_ref[...] * (pre * _CBN) + bt_ref[...]
        o_ref[...] = jax.nn.sigmoid(h3 * wl_ref[...] + bl_ref[...])

    vec = lambda: pl.BlockSpec((_BN, 1), lambda i: (i, 0))
    scl = lambda: pl.BlockSpec((1, 1), lambda i: (0, 0))
    return pl.pallas_call(
        body,
        grid=(n // _BN,),
        in_specs=[vec(), vec(), vec(), vec(), scl(), scl(), scl(), scl(),
                  scl()],
        out_specs=pl.BlockSpec((_BN, 1), lambda i: (i, 0)),
        out_shape=jax.ShapeDtypeStruct((n, 1), jnp.float32),
    )(sp0, sp1, t3, dinv, b3, g3, bt3, wl, bl)


def _pad16(v):
    out = jnp.zeros((1, 16), jnp.float32)
    return out.at[0, : v.shape[0]].set(v)


def kernel(x, edge_index, edge_attr, W1, b1, g1, bt1, W2, b2, g2, bt2,
           W3, b3, g3, bt3, Wl, bl):
    n, _ = x.shape
    e = edge_attr.shape[0]
    chunk = _NW * _SC_EDGES
    e_pad = ((e + chunk - 1) // chunk) * chunk
    pad = e_pad - e
    row = jnp.concatenate([edge_index[0], jnp.zeros((pad,), edge_index.dtype)])
    col = jnp.concatenate([edge_index[1], jnp.zeros((pad,), edge_index.dtype)])
    w = jnp.concatenate(
        [edge_attr.astype(jnp.float32), jnp.zeros((pad,), jnp.float32)])
    row2d = row.reshape(-1, 128)
    col2d = col.reshape(-1, 128)
    z16 = jnp.zeros((n, 16), jnp.float32)
    z1 = jnp.zeros((n,), jnp.float32)

    deg_pass = _make_edge_pass(n, e_pad, "deg")
    vec_pass = _make_edge_pass(n, e_pad, "vec16")
    flat_pass = _make_edge_pass(n, e_pad, "flat1")

    degp = deg_pass(col2d, w, z1)
    p0 = degp[:n, None]
    p1 = degp[n:, None]
    t1, u1, dinv = _tc_first(x, W1, p0, p1)

    s1 = vec_pass(u1, row2d, col2d, w, z16)
    w2p = jnp.zeros((16, 16), jnp.float32).at[:, : W2.shape[1]].set(W2)
    t2, u2 = _tc_mid(s1[:n], s1[n:], t1, dinv,
                     _pad16(b1), _pad16(g1), _pad16(bt1), w2p)

    s2 = vec_pass(u2, row2d, col2d, w, z16)
    w3p = (jnp.zeros((16, 16), jnp.float32)
           .at[: W3.shape[0], : W3.shape[1]].set(W3))
    t3, u3 = _tc_mid(s2[:n], s2[n:], t2, dinv,
                     _pad16(b2), _pad16(g2), _pad16(bt2), w3p)

    s3 = flat_pass(u3[:, 0], row, col2d, w, z1)
    return _tc_final(s3[:n, None], s3[n:, None], t3[:, :1], dinv,
                     _pad16(b3)[:, :1], _pad16(g3)[:, :1], _pad16(bt3)[:, :1],
                     Wl, bl[None, :])


# speedup vs baseline: 19.3987x; 19.3987x over previous
import functools
import math

import jax
import jax.numpy as jnp
from jax import lax
from jax.experimental import pallas as pl
from jax.experimental.pallas import tpu as pltpu
from jax.experimental.pallas import tpu_sc as plsc

_NC = 2
_NS = 16
_NW = _NC * _NS
_KROWS = 16
_SC_EDGES = _KROWS * 128
_CHUNK = _SC_EDGES


@functools.lru_cache(maxsize=None)
def _make_edge_pass(n, e_pad, mode):
    nsup = e_pad // (_NW * _SC_EDGES)
    t_edges = nsup * _SC_EDGES
    nch = (n + _CHUNK - 1) // _CHUNK
    n_pad = nch * _CHUNK
    kd = (nch + _NS - 1) // _NS
    acc_shape = (n_pad,)
    out_shape = (_NC * n_pad,)

    mesh = plsc.VectorSubcoreMesh(core_axis_name="c", subcore_axis_name="s")

    scratch = []
    if mode == "flat1":
        scratch.append(pltpu.VMEM((n,), jnp.float32))
        scratch.append(pltpu.VMEM((_SC_EDGES,), jnp.int32))
    scratch.append(pltpu.VMEM((_SC_EDGES,), jnp.int32))
    scratch.append(pltpu.VMEM((_SC_EDGES,), jnp.float32))
    if mode != "deg":
        scratch.append(pltpu.VMEM((_SC_EDGES,), jnp.float32))
    scratch.append(pltpu.VMEM_SHARED(acc_shape, jnp.float32))
    scratch.append(pltpu.SemaphoreType.DMA)
    scratch.append(pltpu.SemaphoreType.DMA)

    def body(*refs):
        it = iter(refs)
        u_hbm = row_hbm = None
        if mode == "flat1":
            u_hbm = next(it)
            row_hbm = next(it)
        col_hbm = next(it)
        w_hbm = next(it)
        out_hbm = next(it)
        u_v = idx_row = None
        if mode == "flat1":
            u_v = next(it)
            idx_row = next(it)
        idx_col = next(it)
        w_v = next(it)
        msg_v = next(it) if mode != "deg" else None
        acc = next(it)
        gsem = next(it)
        ssem = next(it)

        c = lax.axis_index("c")
        s = lax.axis_index("s")
        g = c * _NS + s

        zbuf = msg_v if mode != "deg" else w_v

        @pl.loop(0, _CHUNK // 16, unroll=8)
        def _zb(q):
            zbuf[pl.ds(q * 16, 16)] = jnp.zeros((16,), jnp.float32)
        for k in range(kd):
            i = s + _NS * k

            @pl.when(i < nch)
            def _():
                off = pl.multiple_of(i * _CHUNK, _CHUNK)
                pltpu.sync_copy(zbuf, acc.at[pl.ds(off, _CHUNK)])

        if mode == "flat1":
            pltpu.sync_copy(u_hbm, u_v)
        plsc.subcore_barrier()

        @pl.loop(0, nsup)
        def _sup(i):
            e_base = pl.multiple_of(g * t_edges + i * _SC_EDGES, _SC_EDGES)
            pltpu.sync_copy(col_hbm.at[pl.ds(e_base, _SC_EDGES)], idx_col)
            pltpu.sync_copy(w_hbm.at[pl.ds(e_base, _SC_EDGES)], w_v)
            if mode == "flat1":
                pltpu.sync_copy(row_hbm.at[pl.ds(e_base, _SC_EDGES)],
                                idx_row)

                @pl.loop(0, _SC_EDGES // 16, unroll=8)
                def _grp(q):
                    idx16 = idx_row[pl.ds(q * 16, 16)]
                    rows16 = plsc.load_gather(u_v, [idx16])
                    msg_v[pl.ds(q * 16, 16)] = (
                        rows16 * w_v[pl.ds(q * 16, 16)])

            src = w_v if mode == "deg" else msg_v
            pltpu.async_copy(src, acc.at[idx_col], ssem, add=True).wait()

        plsc.subcore_barrier()

        for k in range(kd):
            i = s + _NS * k

            @pl.when(i < nch)
            def _():
                off = pl.multiple_of(i * _CHUNK, _CHUNK)
                doff = pl.multiple_of(c * n_pad + i * _CHUNK, _CHUNK)
                pltpu.sync_copy(acc.at[pl.ds(off, _CHUNK)], zbuf)
                pltpu.sync_copy(zbuf, out_hbm.at[pl.ds(doff, _CHUNK)])

    return pl.kernel(
        body,
        out_type=jax.ShapeDtypeStruct(out_shape, jnp.float32),
        mesh=mesh,
        scratch_types=scratch,
        compiler_params=pltpu.CompilerParams(needs_layout_passes=False))


_BN = 1000
_CBN = 1.0 / math.sqrt(1.0 + 1e-5)


def _tc_first(x, w1, p0, p1):
    n, f = x.shape

    def body(x_ref, w_ref, p0_ref, p1_ref, t_ref, u_ref, dinv_ref):
        deg = p0_ref[...] + p1_ref[...] + 1.0
        dinv = lax.rsqrt(deg)
        t = jnp.dot(x_ref[...], w_ref[...],
                    preferred_element_type=jnp.float32)
        t_ref[...] = t
        u_ref[...] = t * dinv
        dinv_ref[...] = dinv

    return pl.pallas_call(
        body,
        grid=(n // _BN,),
        in_specs=[
            pl.BlockSpec((_BN, f), lambda i: (i, 0)),
            pl.BlockSpec((f, 16), lambda i: (0, 0)),
            pl.BlockSpec((_BN, 1), lambda i: (i, 0)),
            pl.BlockSpec((_BN, 1), lambda i: (i, 0)),
        ],
        out_specs=[
            pl.BlockSpec((_BN, 16), lambda i: (i, 0)),
            pl.BlockSpec((_BN, 16), lambda i: (i, 0)),
            pl.BlockSpec((_BN, 1), lambda i: (i, 0)),
        ],
        out_shape=[
            jax.ShapeDtypeStruct((n, 16), jnp.float32),
            jax.ShapeDtypeStruct((n, 16), jnp.float32),
            jax.ShapeDtypeStruct((n, 1), jnp.float32),
        ],
    )(x, w1, p0, p1)


def _tc_mid(sp0, sp1, t, dinv, b, gam, bt, wnext):
    n = t.shape[0]

    def body(p0_ref, p1_ref, t_ref, dinv_ref, b_ref, g_ref, bt_ref, w_ref,
             tn_ref, un_ref):
        dinv = dinv_ref[...]
        pre = (dinv * (p0_ref[...] + p1_ref[...])
               + (dinv * dinv) * t_ref[...] + b_ref[...])
        h = jnp.maximum(g_ref[...] * (pre * _CBN) + bt_ref[...], 0.0)
        tn = jnp.dot(h, w_ref[...], preferred_element_type=jnp.float32)
        tn_ref[...] = tn
        un_ref[...] = tn * dinv

    return pl.pallas_call(
        body,
        grid=(n // _BN,),
        in_specs=[
            pl.BlockSpec((_BN, 16), lambda i: (i, 0)),
            pl.BlockSpec((_BN, 16), lambda i: (i, 0)),
            pl.BlockSpec((_BN, 16), lambda i: (i, 0)),
            pl.BlockSpec((_BN, 1), lambda i: (i, 0)),
            pl.BlockSpec((1, 16), lambda i: (0, 0)),
            pl.BlockSpec((1, 16), lambda i: (0, 0)),
            pl.BlockSpec((1, 16), lambda i: (0, 0)),
            pl.BlockSpec((16, 16), lambda i: (0, 0)),
        ],
        out_specs=[
            pl.BlockSpec((_BN, 16), lambda i: (i, 0)),
            pl.BlockSpec((_BN, 16), lambda i: (i, 0)),
        ],
        out_shape=[
            jax.ShapeDtypeStruct((n, 16), jnp.float32),
            jax.ShapeDtypeStruct((n, 16), jnp.float32),
        ],
    )(sp0, sp1, t, dinv, b, gam, bt, wnext)


def _tc_final(sp0, sp1, t3, dinv, b3, g3, bt3, wl, bl):
    n = t3.shape[0]

    def body(p0_ref, p1_ref, t_ref, dinv_ref, b_ref, g_ref, bt_ref,
             wl_ref, bl_ref, o_ref):
        dinv = dinv_ref[...]
        pre = (dinv * (p0_ref[...] + p1_ref[...])
               + (dinv * dinv) * t_ref[...] + b_ref[...])
        h3 = g_ref[...] * (pre * _CBN) + bt_ref[...]
        o_ref[...] = jax.nn.sigmoid(h3 * wl_ref[...] + bl_ref[...])

    vec = lambda: pl.BlockSpec((_BN, 1), lambda i: (i, 0))
    scl = lambda: pl.BlockSpec((1, 1), lambda i: (0, 0))
    return pl.pallas_call(
        body,
        grid=(n // _BN,),
        in_specs=[vec(), vec(), vec(), vec(), scl(), scl(), scl(), scl(),
                  scl()],
        out_specs=pl.BlockSpec((_BN, 1), lambda i: (i, 0)),
        out_shape=jax.ShapeDtypeStruct((n, 1), jnp.float32),
    )(sp0, sp1, t3, dinv, b3, g3, bt3, wl, bl)


def _pad16(v):
    out = jnp.zeros((1, 16), jnp.float32)
    return out.at[0, : v.shape[0]].set(v)


def kernel(x, edge_index, edge_attr, W1, b1, g1, bt1, W2, b2, g2, bt2,
           W3, b3, g3, bt3, Wl, bl):
    n, _ = x.shape
    e = edge_attr.shape[0]
    chunk = _NW * _SC_EDGES
    e_pad = ((e + chunk - 1) // chunk) * chunk
    pad = e_pad - e
    nch = (n + _CHUNK - 1) // _CHUNK
    n_pad = nch * _CHUNK
    row = jnp.concatenate([edge_index[0], jnp.zeros((pad,), edge_index.dtype)])
    col = jnp.concatenate([edge_index[1], jnp.zeros((pad,), edge_index.dtype)])
    w = jnp.concatenate(
        [edge_attr.astype(jnp.float32), jnp.zeros((pad,), jnp.float32)])

    deg_pass = _make_edge_pass(n, e_pad, "deg")
    flat_pass = _make_edge_pass(n, e_pad, "flat1")

    def prop16(u, ncols):
        cols = [flat_pass(u[:, k], row, col, w) for k in range(ncols)]
        z = jnp.zeros((n,), jnp.float32)
        p0 = jnp.stack([s[:n] for s in cols]
                       + [z] * (16 - ncols), axis=1)
        p1 = jnp.stack([s[n_pad:n_pad + n] for s in cols]
                       + [z] * (16 - ncols), axis=1)
        return p0, p1

    degp = deg_pass(col, w)
    p0 = degp[:n, None]
    p1 = degp[n_pad:n_pad + n, None]
    t1, u1, dinv = _tc_first(x, W1, p0, p1)

    s1a, s1b = prop16(u1, 16)
    w2p = jnp.zeros((16, 16), jnp.float32).at[:, : W2.shape[1]].set(W2)
    t2, u2 = _tc_mid(s1a, s1b, t1, dinv,
                     _pad16(b1), _pad16(g1), _pad16(bt1), w2p)

    s2a, s2b = prop16(u2, W2.shape[1])
    w3p = (jnp.zeros((16, 16), jnp.float32)
           .at[: W3.shape[0], : W3.shape[1]].set(W3))
    t3, u3 = _tc_mid(s2a, s2b, t2, dinv,
                     _pad16(b2), _pad16(g2), _pad16(bt2), w3p)

    s3 = flat_pass(u3[:, 0], row, col, w)
    return _tc_final(s3[:n, None], s3[n_pad:n_pad + n, None], t3[:, :1], dinv,
                     _pad16(b3)[:, :1], _pad16(g3)[:, :1], _pad16(bt3)[:, :1],
                     Wl, bl[None, :])
